# Initial kernel scaffold; baseline (speedup 1.0000x reference)
import jax
import jax.numpy as jnp
from jax.experimental import pallas as pl


def kernel(features, edge_index, edge_weight, edge_index2, edge_weight2, ln_W, ln_b, W1, b1, W2, b2, bn_gamma, bn_beta):
    raise NotImplementedError("write your pallas kernel here")


# speedup vs baseline: 4.6559x; 4.6559x over previous
import functools

import jax
import jax.numpy as jnp
from jax import lax
from jax.experimental import pallas as pl
from jax.experimental.pallas import tpu as pltpu
from jax.experimental.pallas import tpu_sc as plsc

N = 10000
D = 128
C = 64
E = 320000

NS = 16
EC = 128
NR = E // EC
ZR = 624
TAIL = N - NS * ZR

_MB = 1000
_GRID = N // _MB


def _mm_body(x_ref, w_ref, o1_ref, o2_ref):
    xw = jnp.dot(x_ref[...], w_ref[...], preferred_element_type=jnp.float32)
    o1_ref[...] = xw[:, :C]
    o2_ref[...] = xw[:, C:]


def _matmul2(features, w12):
    return pl.pallas_call(
        _mm_body,
        grid=(_GRID,),
        in_specs=[pl.BlockSpec((_MB, D), lambda i: (i, 0)),
                  pl.BlockSpec((D, 2 * C), lambda i: (0, 0))],
        out_specs=[pl.BlockSpec((_MB, C), lambda i: (i, 0)),
                   pl.BlockSpec((_MB, C), lambda i: (i, 0))],
        out_shape=[jax.ShapeDtypeStruct((N, C), jnp.float32),
                   jax.ShapeDtypeStruct((N, C), jnp.float32)],
    )(features, w12)


def _combine_body(x_ref, w_ref, a0_ref, a1_ref, b_ref, o_ref):
    o_ref[...] = (jnp.dot(x_ref[...], w_ref[...], preferred_element_type=jnp.float32)
                  + a0_ref[...] + a1_ref[...] + b_ref[0:1, :])


def _combine(features, lnw, acc, bias8):
    return pl.pallas_call(
        _combine_body,
        grid=(_GRID,),
        in_specs=[pl.BlockSpec((_MB, D), lambda i: (i, 0)),
                  pl.BlockSpec((D, C), lambda i: (0, 0)),
                  pl.BlockSpec((_MB, C), lambda i: (i, 0)),
                  pl.BlockSpec((_MB, C), lambda i: (i + _GRID, 0)),
                  pl.BlockSpec((8, C), lambda i: (0, 0))],
        out_specs=pl.BlockSpec((_MB, C), lambda i: (i, 0)),
        out_shape=jax.ShapeDtypeStruct((N, C), jnp.float32),
    )(features, lnw, acc, acc, bias8)


_sc_mesh = plsc.VectorSubcoreMesh(core_axis_name="c", subcore_axis_name="s")


@functools.partial(
    pl.kernel,
    out_type=jax.ShapeDtypeStruct((2 * N, C), jnp.float32),
    mesh=_sc_mesh,
    scratch_types=[
        pltpu.VMEM((EC,), jnp.int32),
        pltpu.VMEM((EC,), jnp.int32),
        pltpu.VMEM((EC,), jnp.float32),
        pltpu.VMEM((EC, C), jnp.float32),
        pltpu.VMEM_SHARED((N, C), jnp.float32),
        pltpu.SemaphoreType.DMA,
    ],
    compiler_params=pltpu.CompilerParams(needs_layout_passes=False,
                                         use_tc_tiling_on_sc=False),
)
def _sc_edges(xw1, xw2, src1, dst1, w1, src2, dst2, w2,
              out, src_v, dst_v, w_v, rows_v, acc, sem):
    c = lax.axis_index("c")
    s = lax.axis_index("s")

    zero16 = jnp.zeros((16,), jnp.float32)

    def _zrow(i, carry):
        for j in range(C // 16):
            rows_v[i, pl.ds(j * 16, 16)] = zero16
        return carry

    lax.fori_loop(0, EC, _zrow, 0)
    for k in range(ZR // EC):
        pltpu.sync_copy(rows_v, acc.at[pl.ds(s * ZR + k * EC, EC)])
    rem = ZR % EC
    if rem:
        pltpu.sync_copy(rows_v.at[pl.ds(0, rem)],
                        acc.at[pl.ds(s * ZR + (ZR // EC) * EC, rem)])

    @pl.when(s == NS - 1)
    def _():
        pltpu.sync_copy(rows_v.at[pl.ds(0, TAIL)],
                        acc.at[pl.ds(NS * ZR, TAIL)])

    plsc.subcore_barrier()

    base_cnt = NR // NS
    extra = NR - base_cnt * NS
    r0 = s * base_cnt + jnp.minimum(s, extra)
    cnt = base_cnt + (s < extra).astype(jnp.int32)

    def _process(xw, src_h, dst_h, w_h):
        def _chunk(r, carry):
            base = r * EC
            pltpu.sync_copy(src_h.at[pl.ds(base, EC)], src_v)
            pltpu.sync_copy(dst_h.at[pl.ds(base, EC)], dst_v)
            pltpu.sync_copy(w_h.at[pl.ds(base, EC)], w_v)
            pltpu.async_copy(xw.at[src_v], rows_v, sem).wait()

            def _mul(e, mc):
                wv = plsc.load_gather(w_v, [jnp.full((16,), e, jnp.int32)])
                for j in range(C // 16):
                    rows_v[e, pl.ds(j * 16, 16)] = rows_v[e, pl.ds(j * 16, 16)] * wv
                return mc

            lax.fori_loop(0, EC, _mul, 0)
            pltpu.sync_copy(rows_v, acc.at[dst_v], add=True)
            return carry

        lax.fori_loop(r0, r0 + cnt, _chunk, 0)

    @pl.when(c == 0)
    def _():
        _process(xw1, src1, dst1, w1)

    @pl.when(c == 1)
    def _():
        _process(xw2, src2, dst2, w2)

    plsc.subcore_barrier()
    pltpu.sync_copy(acc.at[pl.ds(s * ZR, ZR)],
                    out.at[pl.ds(c * N + s * ZR, ZR)])

    @pl.when(s == NS - 1)
    def _():
        pltpu.sync_copy(acc.at[pl.ds(NS * ZR, TAIL)],
                        out.at[pl.ds(c * N + NS * ZR, TAIL)])


def kernel(features, edge_index, edge_weight, edge_index2, edge_weight2,
           ln_W, ln_b, W1, b1, W2, b2, bn_gamma, bn_beta):
    scale = bn_gamma * lax.rsqrt(jnp.float32(1.0 + 1e-5))
    w12 = jnp.concatenate([W1 * scale[None, :], W2 * scale[None, :]], axis=1)
    lnw = ln_W * scale[None, :]
    bias8 = jnp.tile(((ln_b + b1 + b2) * scale + bn_beta)[None, :], (8, 1))

    xw1, xw2 = _matmul2(features, w12)
    acc = _sc_edges(xw1, xw2,
                    edge_index[0], edge_index[1], edge_weight,
                    edge_index2[0], edge_index2[1], edge_weight2)
    return _combine(features, lnw, acc, bias8)


# speedup vs baseline: 7.0488x; 1.5139x over previous
import functools

import jax
import jax.numpy as jnp
from jax import lax
from jax.experimental import pallas as pl
from jax.experimental.pallas import tpu as pltpu
from jax.experimental.pallas import tpu_sc as plsc

N = 10000
D = 128
C = 64
E = 320000

NS = 16
EC = 128
CPT = 160
EPAD = NS * CPT * EC
K = 4
NB = CPT // K
NSUPER = NB // 2
BE = K * EC

ZR = 624
TAIL = N - NS * ZR

_MB = 1000
_GRID = N // _MB


def _mm_body(x_ref, w_ref, o1_ref, o2_ref):
    xw = jnp.dot(x_ref[...], w_ref[...], preferred_element_type=jnp.float32)
    o1_ref[...] = xw[:, :C]
    o2_ref[...] = xw[:, C:]


def _matmul2(features, w12):
    return pl.pallas_call(
        _mm_body,
        grid=(_GRID,),
        in_specs=[pl.BlockSpec((_MB, D), lambda i: (i, 0)),
                  pl.BlockSpec((D, 2 * C), lambda i: (0, 0))],
        out_specs=[pl.BlockSpec((_MB, C), lambda i: (i, 0)),
                   pl.BlockSpec((_MB, C), lambda i: (i, 0))],
        out_shape=[jax.ShapeDtypeStruct((N, C), jnp.float32),
                   jax.ShapeDtypeStruct((N, C), jnp.float32)],
    )(features, w12)


def _combine_body(x_ref, w_ref, a0_ref, a1_ref, b_ref, o_ref):
    o_ref[...] = (jnp.dot(x_ref[...], w_ref[...], preferred_element_type=jnp.float32)
                  + a0_ref[...] + a1_ref[...] + b_ref[0:1, :])


def _combine(features, lnw, acc, bias8):
    return pl.pallas_call(
        _combine_body,
        grid=(_GRID,),
        in_specs=[pl.BlockSpec((_MB, D), lambda i: (i, 0)),
                  pl.BlockSpec((D, C), lambda i: (0, 0)),
                  pl.BlockSpec((_MB, C), lambda i: (i, 0)),
                  pl.BlockSpec((_MB, C), lambda i: (i + _GRID, 0)),
                  pl.BlockSpec((8, C), lambda i: (0, 0))],
        out_specs=pl.BlockSpec((_MB, C), lambda i: (i, 0)),
        out_shape=jax.ShapeDtypeStruct((N, C), jnp.float32),
    )(features, lnw, acc, acc, bias8)


_sc_mesh = plsc.VectorSubcoreMesh(core_axis_name="c", subcore_axis_name="s")


@functools.partial(
    pl.kernel,
    out_type=jax.ShapeDtypeStruct((2 * N, C), jnp.float32),
    mesh=_sc_mesh,
    scratch_types=[
        pltpu.VMEM((2, K, EC), jnp.int32),
        pltpu.VMEM((2, K, EC), jnp.int32),
        pltpu.VMEM((2, K, EC), jnp.float32),
        pltpu.VMEM((BE, C), jnp.float32),
        pltpu.VMEM((BE, C), jnp.float32),
        pltpu.VMEM_SHARED((N, C), jnp.float32),
        pltpu.SemaphoreType.DMA,
        pltpu.SemaphoreType.DMA,
        pltpu.SemaphoreType.DMA,
    ],
    compiler_params=pltpu.CompilerParams(needs_layout_passes=False,
                                         use_tc_tiling_on_sc=False),
)
def _sc_edges(xw1, xw2, src1, dst1, w1, src2, dst2, w2,
              out, srcs_v, dsts_v, ws_v, rows_a, rows_b, acc,
              gsem_a, gsem_b, isem):
    c = lax.axis_index("c")
    s = lax.axis_index("s")

    zero16 = jnp.zeros((16,), jnp.float32)

    def _zrow(i, carry):
        for j in range(C // 16):
            rows_a[i, pl.ds(j * 16, 16)] = zero16
        return carry

    lax.fori_loop(0, EC, _zrow, 0)
    for k in range(ZR // EC):
        pltpu.sync_copy(rows_a.at[pl.ds(0, EC)], acc.at[pl.ds(s * ZR + k * EC, EC)])
    pltpu.sync_copy(rows_a.at[pl.ds(0, ZR % EC)],
                    acc.at[pl.ds(s * ZR + (ZR // EC) * EC, ZR % EC)])

    @pl.when(s == NS - 1)
    def _():
        pltpu.sync_copy(rows_a.at[pl.ds(0, TAIL)], acc.at[pl.ds(NS * ZR, TAIL)])

    plsc.subcore_barrier()

    def _process(xw, src_h, dst_h, w_h):
        def load_idx(r, half):
            row = s * CPT + r * K
            pltpu.async_copy(src_h.at[pl.ds(row, K)], srcs_v.at[half], isem)
            pltpu.async_copy(dst_h.at[pl.ds(row, K)], dsts_v.at[half], isem)
            pltpu.async_copy(w_h.at[pl.ds(row, K)], ws_v.at[half], isem)
            pltpu.make_async_copy(src_h.at[pl.ds(0, K)], srcs_v.at[half], isem).wait()
            pltpu.make_async_copy(dst_h.at[pl.ds(0, K)], dsts_v.at[half], isem).wait()
            pltpu.make_async_copy(w_h.at[pl.ds(0, K)], ws_v.at[half], isem).wait()

        def fire_gathers(half, buf, sem):
            for kk in range(K):
                pltpu.async_copy(xw.at[srcs_v.at[half, kk]],
                                 buf.at[pl.ds(kk * EC, EC)], sem)

        def drain_gathers(buf, sem):
            pltpu.make_async_copy(xw.at[pl.ds(0, BE)], buf, sem).wait()

        def mul_scatter(half, buf):
            @plsc.parallel_loop(0, BE, unroll=4)
            def _(e):
                kk = lax.shift_right_logical(e, 7)
                col = lax.bitwise_and(e, EC - 1)
                wv = plsc.load_gather(
                    ws_v, [jnp.full((16,), half, jnp.int32),
                           jnp.full((16,), kk, jnp.int32),
                           jnp.full((16,), col, jnp.int32)])
                for j in range(C // 16):
                    buf[e, pl.ds(j * 16, 16)] = buf[e, pl.ds(j * 16, 16)] * wv

            for kk in range(K):
                pltpu.sync_copy(buf.at[pl.ds(kk * EC, EC)],
                                acc.at[dsts_v.at[half, kk]], add=True)

        load_idx(0, 0)
        fire_gathers(0, rows_a, gsem_a)

        def _super(i, carry):
            load_idx(2 * i + 1, 1)
            fire_gathers(1, rows_b, gsem_b)
            drain_gathers(rows_a, gsem_a)
            mul_scatter(0, rows_a)

            @pl.when(i < NSUPER - 1)
            def _():
                load_idx(2 * i + 2, 0)
                fire_gathers(0, rows_a, gsem_a)

            drain_gathers(rows_b, gsem_b)
            mul_scatter(1, rows_b)
            return carry

        lax.fori_loop(0, NSUPER, _super, 0)

    @pl.when(c == 0)
    def _():
        _process(xw1, src1, dst1, w1)

    @pl.when(c == 1)
    def _():
        _process(xw2, src2, dst2, w2)

    plsc.subcore_barrier()
    pltpu.sync_copy(acc.at[pl.ds(s * ZR, ZR)],
                    out.at[pl.ds(c * N + s * ZR, ZR)])

    @pl.when(s == NS - 1)
    def _():
        pltpu.sync_copy(acc.at[pl.ds(NS * ZR, TAIL)],
                        out.at[pl.ds(c * N + NS * ZR, TAIL)])


def _pad_chunks(a, fill):
    pad = jnp.full((EPAD - E,), fill, a.dtype)
    return jnp.concatenate([a, pad]).reshape(NS * CPT, EC)


def kernel(features, edge_index, edge_weight, edge_index2, edge_weight2,
           ln_W, ln_b, W1, b1, W2, b2, bn_gamma, bn_beta):
    scale = bn_gamma * lax.rsqrt(jnp.float32(1.0 + 1e-5))
    w12 = jnp.concatenate([W1 * scale[None, :], W2 * scale[None, :]], axis=1)
    lnw = ln_W * scale[None, :]
    bias8 = jnp.tile(((ln_b + b1 + b2) * scale + bn_beta)[None, :], (8, 1))

    src1 = _pad_chunks(edge_index[0], 0)
    dst1 = _pad_chunks(edge_index[1], 0)
    w1e = _pad_chunks(edge_weight, 0.0)
    src2 = _pad_chunks(edge_index2[0], 0)
    dst2 = _pad_chunks(edge_index2[1], 0)
    w2e = _pad_chunks(edge_weight2, 0.0)

    xw1, xw2 = _matmul2(features, w12)
    acc = _sc_edges(xw1, xw2, src1, dst1, w1e, src2, dst2, w2e)
    return _combine(features, lnw, acc, bias8)
